# baseline (speedup 1.0000x reference)
import functools

import jax
import jax.numpy as jnp
from jax import lax
from jax.experimental import pallas as pl
from jax.experimental.pallas import tpu as pltpu
from jax.experimental.pallas import tpu_sc as plsc

N = 10000
E = 160000
H = 256
DE = 16
NB = 16
GFC = 32
OUT = 4

NC = 2
NT = 16
K = 80
EPT = E // NT
NCH = EPT // K
RPT = N // NT

DW = 640
DPT = DW // NT

RB = 400
NRB = N // RB
EB = 3200
NEB = E // EB

_F32 = jnp.float32
_HIGH = jax.lax.Precision.HIGHEST


def _dot(a, b):
    return jnp.dot(a, b, precision=_HIGH, preferred_element_type=_F32)


def _make_pre(din_parts):
    nparts = len(din_parts)

    def body(*refs):
        x_refs = refs[:nparts]
        w_ref, a_ref, ha_ref, hb_ref, hsd_ref = refs[nparts:]
        c = pl.program_id(1)
        x = jnp.concatenate([r[...] for r in x_refs], axis=1) if nparts > 1 else x_refs[0][...]
        hh = _dot(x, w_ref[...])
        ha_ref[...] = hh[:, :64]
        hb_ref[...] = hh[:, 64:]
        contrib = _dot(hh, a_ref[...])

        @pl.when(c == 0)
        def _():
            hsd_ref[...] = contrib

        @pl.when(c > 0)
        def _():
            hsd_ref[...] += contrib

    din = sum(din_parts)
    in_specs = [pl.BlockSpec((RB, p), (lambda i, c: (i, 0))) for p in din_parts]
    in_specs += [
        pl.BlockSpec((din, 128), lambda i, c: (0, c)),
        pl.BlockSpec((128, 128), lambda i, c: (c, 0)),
    ]
    return pl.pallas_call(
        body,
        grid=(NRB, NC),
        in_specs=in_specs,
        out_specs=[
            pl.BlockSpec((RB, 64), lambda i, c: (c * NRB + i, 0)),
            pl.BlockSpec((RB, 64), lambda i, c: (c * NRB + i, 0)),
            pl.BlockSpec((RB, 128), lambda i, c: (i, 0)),
        ],
        out_shape=[
            jax.ShapeDtypeStruct((2 * N, 64), _F32),
            jax.ShapeDtypeStruct((2 * N, 64), _F32),
            jax.ShapeDtypeStruct((N, 128), _F32),
        ],
    )


def _et_body(ea_ref, we_ref, ae_ref, out_ref):
    w = _dot(we_ref[...], ae_ref[...])
    etb = _dot(ea_ref[...], w)
    out_ref[...] = etb.reshape(1, EB // 128, 128)


_et_call = pl.pallas_call(
    _et_body,
    grid=(NEB,),
    in_specs=[
        pl.BlockSpec((EB, DE), lambda i: (i, 0)),
        pl.BlockSpec((DE, H), lambda i: (0, 0)),
        pl.BlockSpec((H, 1), lambda i: (0, 0)),
    ],
    out_specs=pl.BlockSpec((1, EB // 128, 128), lambda i: (i, 0, 0)),
    out_shape=jax.ShapeDtypeStruct((NEB, EB // 128, 128), _F32),
)


def _sc_gat(h4a, h4b, s2, d2, et2, hs, hd):
    mesh = plsc.VectorSubcoreMesh(
        core_axis_name="c", subcore_axis_name="s", num_cores=NC, num_subcores=NT)

    @functools.partial(
        pl.kernel,
        out_type=[
            jax.ShapeDtypeStruct((2 * NC * NT, RPT, 64), _F32),
            jax.ShapeDtypeStruct((NC * NT, RPT, 16), _F32),
        ],
        mesh=mesh,
        compiler_params=pltpu.CompilerParams(
            needs_layout_passes=False, use_tc_tiling_on_sc=False),
        scratch_types=[
            pltpu.VMEM((NCH, K), jnp.int32),
            pltpu.VMEM((NCH, K), jnp.int32),
            pltpu.VMEM((NCH, K), _F32),
            pltpu.VMEM((N,), _F32),
            pltpu.VMEM((N,), _F32),
            pltpu.VMEM((K, 64), _F32),
            pltpu.VMEM((K, 64), _F32),
            pltpu.VMEM((K, 16), _F32),
            pltpu.VMEM((RPT, 16), _F32),
            pltpu.VMEM((125, 64), _F32),
            pltpu.VMEM_SHARED((N, 64), _F32),
            pltpu.VMEM_SHARED((N, 16), _F32),
            pltpu.SemaphoreType.DMA,
            pltpu.SemaphoreType.DMA,
            pltpu.SemaphoreType.DMA,
            pltpu.SemaphoreType.DMA,
        ],
    )
    def k(h4a_hbm, h4b_hbm, s2_hbm, d2_hbm, et2_hbm, hs_hbm, hd_hbm, onum, oden,
          svm, dvm, exvm, hsv, hdv, rows0, rows1, dr, zbufd, zbuf, acc, dacc,
          semg0, semg1, sems0, sems1):
        c = lax.axis_index("c")
        t = lax.axis_index("s")
        qbase = 2 * c

        pltpu.sync_copy(s2_hbm.at[t], svm)
        pltpu.sync_copy(d2_hbm.at[t], dvm)
        pltpu.sync_copy(et2_hbm.at[t], exvm)
        pltpu.sync_copy(hs_hbm, hsv)
        pltpu.sync_copy(hd_hbm, hdv)

        z16 = jnp.zeros((16,), _F32)
        iot = lax.iota(jnp.int32, 16)

        def zb(i, carry):
            for u in range(4):
                zbuf[i, pl.ds(u * 16, 16)] = z16
            return carry

        lax.fori_loop(0, 125, zb, 0)

        def zbd(i, carry):
            zbufd[i, :] = z16
            return carry

        lax.fori_loop(0, RPT, zbd, 0)

        def zero_acc():
            for r in range(5):
                pltpu.sync_copy(zbuf, acc.at[pl.ds(t * RPT + r * 125, 125)])

        zero_acc()
        pltpu.sync_copy(zbufd, dacc.at[pl.ds(t * RPT, RPT)])

        def p1(ci, carry):
            for u in range(K // 16):
                sl = pl.ds(u * 16, 16)
                sv = svm[ci, sl]
                dv = dvm[ci, sl]
                hsg = plsc.load_gather(hsv, [sv])
                hdg = plsc.load_gather(hdv, [dv])
                al = hsg + hdg + exvm[ci, sl]
                al = jnp.where(al >= 0.0, al, al * 0.2)
                exvm[ci, sl] = jnp.exp(al)
                svm[ci, sl] = sv + c * N
            return carry

        lax.fori_loop(0, NCH, p1, 0)
        plsc.subcore_barrier()

        bufs = ((rows0, semg0, sems0), (rows1, semg1, sems1))

        def run_pass(h_hbm, first):
            def process(ci, rows, sems):
                if first:
                    for u in range(K // 16):
                        ev = exvm[ci, pl.ds(u * 16, 16)]
                        plsc.store_scatter(dr, [iot + u * 16, iot * 0], ev)
                for j in range(K):
                    exb = plsc.load_gather(
                        exvm,
                        [jnp.full((16,), ci, jnp.int32),
                         jnp.full((16,), j, jnp.int32)],
                    )
                    for u in range(4):
                        sl = pl.ds(u * 16, 16)
                        rows[j, sl] = rows[j, sl] * exb
                pltpu.async_copy(rows, acc.at[dvm.at[ci]], sems, add=True)
                if first:
                    pltpu.sync_copy(dr, dacc.at[dvm.at[ci]], add=True)

            def gwait(rows, sem):
                pltpu.make_async_copy(h_hbm.at[svm.at[0]], rows, sem).wait()

            def swait(rows, sem):
                pltpu.make_async_copy(rows, acc.at[dvm.at[0]], sem).wait()

            pltpu.async_copy(h_hbm.at[svm.at[0]], rows0, semg0)

            def body2(ci, carry):
                for b in range(2):
                    rows, semg, sems = bufs[b]
                    nrows, nsemg, nsems = bufs[1 - b]
                    cb = ci * 2 + b
                    gwait(rows, semg)

                    @pl.when(cb > 0)
                    def _():
                        swait(nrows, nsems)

                    pltpu.async_copy(h_hbm.at[svm.at[cb + 1]], nrows, nsemg)
                    process(cb, rows, sems)
                return carry

            lax.fori_loop(0, (NCH - 1) // 2, body2, 0, unroll=False)
            last = NCH - 1
            gwait(rows0, semg0)
            swait(rows1, sems1)
            process(last, rows0, sems0)
            swait(rows0, sems0)

        run_pass(h4a_hbm, True)
        plsc.subcore_barrier()
        pltpu.sync_copy(acc.at[pl.ds(t * RPT, RPT)], onum.at[qbase * NT + t])
        pltpu.sync_copy(dacc.at[pl.ds(t * RPT, RPT)], oden.at[c * NT + t])
        zero_acc()
        plsc.subcore_barrier()

        run_pass(h4b_hbm, False)
        plsc.subcore_barrier()
        pltpu.sync_copy(acc.at[pl.ds(t * RPT, RPT)], onum.at[(qbase + 1) * NT + t])

    num4, den = k(h4a, h4b, s2, d2, et2, hs, hd)
    num = jnp.transpose(num4.reshape(4, NT, RPT, 64), (1, 2, 0, 3)).reshape(N, H)
    den = den[:NT].reshape(N, 16)[:, 0].reshape(NRB, 1, RB)
    return num, den


def _make_post1(relu):
    def body(num_ref, den_ref, b_ref, o_ref, st_ref):
        i = pl.program_id(0)
        den = den_ref[...].reshape(RB)[:, None]
        o = num_ref[...] / (den + 1e-30) + b_ref[...]
        if relu:
            o = jnp.maximum(o, 0.0)
        o_ref[...] = o
        s0 = jnp.sum(o, axis=0, keepdims=True)
        s1 = jnp.sum(o * o, axis=0, keepdims=True)
        st = jnp.concatenate([s0, s1], axis=0)

        @pl.when(i == 0)
        def _():
            st_ref[...] = st

        @pl.when(i > 0)
        def _():
            st_ref[...] += st

    return pl.pallas_call(
        body,
        grid=(NRB,),
        in_specs=[
            pl.BlockSpec((RB, H), lambda i: (i, 0)),
            pl.BlockSpec((1, 1, RB), lambda i: (i, 0, 0)),
            pl.BlockSpec((1, H), lambda i: (0, 0)),
        ],
        out_specs=[
            pl.BlockSpec((RB, H), lambda i: (i, 0)),
            pl.BlockSpec((2, H), lambda i: (0, 0)),
        ],
        out_shape=[
            jax.ShapeDtypeStruct((N, H), _F32),
            jax.ShapeDtypeStruct((2, H), _F32),
        ],
    )


def _post2_body(o_ref, st_ref, g_ref, be_ref, out_ref):
    s = st_ref[...]
    mu = s[0:1] / N
    var = s[1:2] / N - mu * mu
    inv = g_ref[...] * lax.rsqrt(var + 1e-5)
    out_ref[...] = (o_ref[...] - mu) * inv + be_ref[...]


_post2_call = pl.pallas_call(
    _post2_body,
    grid=(NRB,),
    in_specs=[
        pl.BlockSpec((RB, H), lambda i: (i, 0)),
        pl.BlockSpec((2, H), lambda i: (0, 0)),
        pl.BlockSpec((1, H), lambda i: (0, 0)),
        pl.BlockSpec((1, H), lambda i: (0, 0)),
    ],
    out_specs=pl.BlockSpec((RB, H), lambda i: (i, 0)),
    out_shape=jax.ShapeDtypeStruct((N, H), _F32),
)


def _pool_body(x_ref, bt_ref, gf_ref, w0, b0, w1, b1, w2, b2, w3, b3,
               out_ref, pooled, cnt):
    i = pl.program_id(0)

    @pl.when(i == 0)
    def _():
        pooled[...] = jnp.zeros((NB, H), _F32)
        cnt[...] = jnp.zeros((NB, 128), _F32)

    bt = bt_ref[0, 0, :]
    onehot = (bt[:, None] == lax.iota(jnp.int32, NB)[None, :]).astype(_F32)
    pooled[...] += lax.dot_general(onehot, x_ref[...], (((0,), (0,)), ((), ())),
                                   precision=_HIGH, preferred_element_type=_F32)
    cnt[...] += jnp.sum(onehot, axis=0)[:, None]

    @pl.when(i == NRB - 1)
    def _():
        xg = pooled[...] / jnp.maximum(cnt[:, 0:1], 1.0)
        x = jnp.concatenate([xg, gf_ref[...]], axis=1)
        x = jnp.maximum(_dot(x, w0[...]) + b0[0], 0.0)
        x = jnp.maximum(_dot(x, w1[...]) + b1[0], 0.0)
        x = jnp.maximum(_dot(x, w2[...]) + b2[0], 0.0)
        out_ref[...] = _dot(x, w3[...]) + b3[0]


_pool_call = pl.pallas_call(
    _pool_body,
    grid=(NRB,),
    in_specs=[
        pl.BlockSpec((RB, H), lambda i: (i, 0)),
        pl.BlockSpec((1, 1, RB), lambda i: (i, 0, 0)),
        pl.BlockSpec((NB, GFC), lambda i: (0, 0)),
        pl.BlockSpec((H + GFC, 64), lambda i: (0, 0)),
        pl.BlockSpec((1, 64), lambda i: (0, 0)),
        pl.BlockSpec((64, 16), lambda i: (0, 0)),
        pl.BlockSpec((1, 16), lambda i: (0, 0)),
        pl.BlockSpec((16, 8), lambda i: (0, 0)),
        pl.BlockSpec((1, 8), lambda i: (0, 0)),
        pl.BlockSpec((8, OUT), lambda i: (0, 0)),
        pl.BlockSpec((1, OUT), lambda i: (0, 0)),
    ],
    out_specs=pl.BlockSpec((NB, OUT), lambda i: (0, 0)),
    out_shape=jax.ShapeDtypeStruct((NB, OUT), _F32),
    scratch_shapes=[
        pltpu.VMEM((NB, H), _F32),
        pltpu.VMEM((NB, 128), _F32),
    ],
)


_pre1 = _make_pre([H])
_pre3 = _make_pre([H, H, H])
_post1_relu = _make_post1(True)
_post1_norelu = _make_post1(False)


def _conv(x_parts, s2, d2, ea, W, asrc, adst, We, aedge, b, gamma2, beta2,
          relu, after=None):
    A = jnp.concatenate(
        [asrc[:, None], adst[:, None], jnp.zeros((H, 126), _F32)], axis=1)
    pre = _pre3 if len(x_parts) == 3 else _pre1
    h4a, h4b, hsd = pre(*x_parts, W, A)
    et = _et_call(ea, We, aedge[:, None]).reshape(NT, NCH, K)
    if after is not None:
        et, _ = lax.optimization_barrier((et, after))
    hs = hsd[:, 0]
    hd = hsd[:, 1]
    num, den = _sc_gat(h4a, h4b, s2, d2, et, hs, hd)
    post1 = _post1_relu if relu else _post1_norelu
    o, st = post1(num, den, b[None])
    return _post2_call(o, st, gamma2, beta2), den


def kernel(x_alpha, x_beta, x_theta, edge_index_alpha, edge_index_beta,
           edge_index_theta, edge_attr_alpha, edge_attr_beta, edge_attr_theta,
           batch, global_feature,
           W_alpha, asrc_alpha, adst_alpha, We_alpha, aedge_alpha, b_alpha,
           W_beta, asrc_beta, adst_beta, We_beta, aedge_beta, b_beta,
           W_theta, asrc_theta, adst_theta, We_theta, aedge_theta, b_theta,
           W_combined, asrc_combined, adst_combined, We_combined,
           aedge_combined, b_combined,
           W_final, asrc_final, adst_final, We_final, aedge_final, b_final,
           bn_gamma, bn_beta,
           mlp_W0, mlp_b0, mlp_W1, mlp_b1, mlp_W2, mlp_b2, mlp_W3, mlp_b3):
    gamma2 = bn_gamma[None]
    beta2 = bn_beta[None]

    def split_ei(ei):
        return (ei[0].reshape(NT, NCH, K), ei[1].reshape(NT, NCH, K))

    sa, da = split_ei(edge_index_alpha)
    sb, db = split_ei(edge_index_beta)
    st_, dt = split_ei(edge_index_theta)

    xa, tok = _conv((x_alpha,), sa, da, edge_attr_alpha, W_alpha, asrc_alpha,
                    adst_alpha, We_alpha, aedge_alpha, b_alpha, gamma2, beta2,
                    True)
    xb, tok = _conv((x_beta,), sb, db, edge_attr_beta, W_beta, asrc_beta,
                    adst_beta, We_beta, aedge_beta, b_beta, gamma2, beta2,
                    True, after=tok)
    xt, tok = _conv((x_theta,), st_, dt, edge_attr_theta, W_theta, asrc_theta,
                    adst_theta, We_theta, aedge_theta, b_theta, gamma2, beta2,
                    True, after=tok)
    xf, tok = _conv((xa, xb, xt), sa, da, edge_attr_alpha, W_combined,
                    asrc_combined, adst_combined, We_combined, aedge_combined,
                    b_combined, gamma2, beta2, True, after=tok)
    xo, _ = _conv((xf,), sa, da, edge_attr_alpha, W_final, asrc_final,
                  adst_final, We_final, aedge_final, b_final, gamma2, beta2,
                  False, after=tok)

    batch3 = batch.reshape(NRB, 1, RB)
    return _pool_call(xo, batch3, global_feature, mlp_W0, mlp_b0[None],
                      mlp_W1, mlp_b1[None], mlp_W2, mlp_b2[None],
                      mlp_W3, mlp_b3[None])


# speedup vs baseline: 1.2088x; 1.2088x over previous
import functools

import jax
import jax.numpy as jnp
from jax import lax
from jax.experimental import pallas as pl
from jax.experimental.pallas import tpu as pltpu
from jax.experimental.pallas import tpu_sc as plsc

N = 10000
E = 160000
H = 256
DE = 16
NB = 16
GFC = 32
OUT = 4

NC = 2
NT = 16
K = 80
EPT = E // NT
NCH = EPT // K

RPT = N // NT

RB = 400
NRB = N // RB
EB = 3200
NEB = E // EB

_F32 = jnp.float32
_HIGH = jax.lax.Precision.HIGHEST


def _dot(a, b):
    return jnp.dot(a, b, precision=_HIGH, preferred_element_type=_F32)


def _make_pre(din_parts):
    nparts = len(din_parts)

    def body(*refs):
        x_refs = refs[:nparts]
        w_ref, a_ref, ha_ref, hb_ref, hsd_ref = refs[nparts:]
        c = pl.program_id(1)
        x = jnp.concatenate([r[...] for r in x_refs], axis=1) if nparts > 1 else x_refs[0][...]
        hh = _dot(x, w_ref[...])
        ha_ref[...] = hh[:, :64]
        hb_ref[...] = hh[:, 64:]
        contrib = _dot(hh, a_ref[...])

        @pl.when(c == 0)
        def _():
            hsd_ref[...] = contrib

        @pl.when(c > 0)
        def _():
            hsd_ref[...] += contrib

    din = sum(din_parts)
    in_specs = [pl.BlockSpec((RB, p), (lambda i, c: (i, 0))) for p in din_parts]
    in_specs += [
        pl.BlockSpec((din, 128), lambda i, c: (0, c)),
        pl.BlockSpec((128, 128), lambda i, c: (c, 0)),
    ]
    return pl.pallas_call(
        body,
        grid=(NRB, NC),
        in_specs=in_specs,
        out_specs=[
            pl.BlockSpec((RB, 64), lambda i, c: (c * NRB + i, 0)),
            pl.BlockSpec((RB, 64), lambda i, c: (c * NRB + i, 0)),
            pl.BlockSpec((RB, 128), lambda i, c: (i, 0)),
        ],
        out_shape=[
            jax.ShapeDtypeStruct((2 * N, 64), _F32),
            jax.ShapeDtypeStruct((2 * N, 64), _F32),
            jax.ShapeDtypeStruct((N, 128), _F32),
        ],
    )


def _et_body(ea_ref, we_ref, ae_ref, out_ref):
    w = _dot(we_ref[...], ae_ref[...])
    etb = _dot(ea_ref[...], w)
    out_ref[...] = etb.reshape(1, EB // 128, 128)


_et_call = pl.pallas_call(
    _et_body,
    grid=(NEB,),
    in_specs=[
        pl.BlockSpec((EB, DE), lambda i: (i, 0)),
        pl.BlockSpec((DE, H), lambda i: (0, 0)),
        pl.BlockSpec((H, 1), lambda i: (0, 0)),
    ],
    out_specs=pl.BlockSpec((1, EB // 128, 128), lambda i: (i, 0, 0)),
    out_shape=jax.ShapeDtypeStruct((NEB, EB // 128, 128), _F32),
)


def _sc_gat(h4a, h4b, s2, d2, et2, hs, hd):
    mesh = plsc.VectorSubcoreMesh(
        core_axis_name="c", subcore_axis_name="s", num_cores=NC, num_subcores=NT)

    @functools.partial(
        pl.kernel,
        out_type=[
            jax.ShapeDtypeStruct((2 * NC * NT, RPT, 64), _F32),
            jax.ShapeDtypeStruct((NC * NT, RPT, 16), _F32),
        ],
        mesh=mesh,
        compiler_params=pltpu.CompilerParams(
            needs_layout_passes=False, use_tc_tiling_on_sc=False),
        scratch_types=[
            pltpu.VMEM((NCH, K), jnp.int32),
            pltpu.VMEM((NCH, K), jnp.int32),
            pltpu.VMEM((NCH, K), _F32),
            pltpu.VMEM((N,), _F32),
            pltpu.VMEM((N,), _F32),
            pltpu.VMEM((K, 64), _F32),
            pltpu.VMEM((K, 64), _F32),
            pltpu.VMEM((K, 16), _F32),
            pltpu.VMEM((RPT, 16), _F32),
            pltpu.VMEM((125, 64), _F32),
            pltpu.VMEM_SHARED((N, 64), _F32),
            pltpu.VMEM_SHARED((N, 16), _F32),
            pltpu.SemaphoreType.DMA,
            pltpu.SemaphoreType.DMA,
            pltpu.SemaphoreType.DMA,
            pltpu.SemaphoreType.DMA,
        ],
    )
    def k(h4a_hbm, h4b_hbm, s2_hbm, d2_hbm, et2_hbm, hs_hbm, hd_hbm, onum, oden,
          svm, dvm, exvm, hsv, hdv, rows0, rows1, dr, zbufd, zbuf, acc, dacc,
          semg0, semg1, sems0, sems1):
        c = lax.axis_index("c")
        t = lax.axis_index("s")
        qbase = 2 * c

        pltpu.sync_copy(s2_hbm.at[t], svm)
        pltpu.sync_copy(d2_hbm.at[t], dvm)
        pltpu.sync_copy(et2_hbm.at[t], exvm)
        pltpu.sync_copy(hs_hbm, hsv)
        pltpu.sync_copy(hd_hbm, hdv)

        z16 = jnp.zeros((16,), _F32)
        iot = lax.iota(jnp.int32, 16)

        def zb(i, carry):
            for u in range(4):
                zbuf[i, pl.ds(u * 16, 16)] = z16
            return carry

        lax.fori_loop(0, 125, zb, 0)

        def zbd(i, carry):
            zbufd[i, :] = z16
            return carry

        lax.fori_loop(0, RPT, zbd, 0)

        def zero_acc():
            for r in range(5):
                pltpu.sync_copy(zbuf, acc.at[pl.ds(t * RPT + r * 125, 125)])

        zero_acc()
        pltpu.sync_copy(zbufd, dacc.at[pl.ds(t * RPT, RPT)])

        def p1(ci, carry):
            for u in range(K // 16):
                sl = pl.ds(u * 16, 16)
                sv = svm[ci, sl]
                dv = dvm[ci, sl]
                hsg = plsc.load_gather(hsv, [sv])
                hdg = plsc.load_gather(hdv, [dv])
                al = hsg + hdg + exvm[ci, sl]
                al = jnp.where(al >= 0.0, al, al * 0.2)
                exvm[ci, sl] = jnp.exp(al)
                svm[ci, sl] = sv + c * N
            return carry

        lax.fori_loop(0, NCH, p1, 0)
        plsc.subcore_barrier()

        bufs = ((rows0, semg0, sems0), (rows1, semg1, sems1))

        def run_pass(h_hbm, first):
            def process(ci, rows, sems):
                for u in range(K // 16):
                    ev = exvm[ci, pl.ds(u * 16, 16)]
                    if first:
                        plsc.store_scatter(dr, [iot + u * 16, iot * 0], ev)
                    for jj in range(16):
                        exb = lax.gather(
                            ev, jnp.full((16, 1), jj, jnp.int32),
                            lax.GatherDimensionNumbers(
                                offset_dims=(), collapsed_slice_dims=(0,),
                                start_index_map=(0,)),
                            (1,),
                            mode=lax.GatherScatterMode.PROMISE_IN_BOUNDS)
                        j = u * 16 + jj
                        for q in range(4):
                            sl = pl.ds(q * 16, 16)
                            rows[j, sl] = rows[j, sl] * exb
                pltpu.async_copy(rows, acc.at[dvm.at[ci]], sems, add=True)
                if first:
                    pltpu.sync_copy(dr, dacc.at[dvm.at[ci]], add=True)

            def gwait(rows, sem):
                pltpu.make_async_copy(h_hbm.at[svm.at[0]], rows, sem).wait()

            def swait(rows, sem):
                pltpu.make_async_copy(rows, acc.at[dvm.at[0]], sem).wait()

            pltpu.async_copy(h_hbm.at[svm.at[0]], rows0, semg0)

            def body2(ci, carry):
                for b in range(2):
                    rows, semg, sems = bufs[b]
                    nrows, nsemg, nsems = bufs[1 - b]
                    cb = ci * 2 + b
                    gwait(rows, semg)

                    @pl.when(cb > 0)
                    def _():
                        swait(nrows, nsems)

                    pltpu.async_copy(h_hbm.at[svm.at[cb + 1]], nrows, nsemg)
                    process(cb, rows, sems)
                return carry

            lax.fori_loop(0, (NCH - 1) // 2, body2, 0, unroll=False)
            last = NCH - 1
            gwait(rows0, semg0)
            swait(rows1, sems1)
            process(last, rows0, sems0)
            swait(rows0, sems0)

        run_pass(h4a_hbm, True)
        plsc.subcore_barrier()
        pltpu.sync_copy(acc.at[pl.ds(t * RPT, RPT)], onum.at[qbase * NT + t])
        pltpu.sync_copy(dacc.at[pl.ds(t * RPT, RPT)], oden.at[c * NT + t])
        zero_acc()
        plsc.subcore_barrier()

        run_pass(h4b_hbm, False)
        plsc.subcore_barrier()
        pltpu.sync_copy(acc.at[pl.ds(t * RPT, RPT)], onum.at[(qbase + 1) * NT + t])

    num4, den = k(h4a, h4b, s2, d2, et2, hs, hd)
    num = jnp.transpose(num4.reshape(4, NT, RPT, 64), (1, 2, 0, 3)).reshape(N, H)
    den = den[:NT].reshape(N, 16)[:, 0].reshape(NRB, 1, RB)
    return num, den


def _make_post1(relu):
    def body(num_ref, den_ref, b_ref, o_ref, st_ref):
        i = pl.program_id(0)
        den = den_ref[...].reshape(RB)[:, None]
        o = num_ref[...] / (den + 1e-30) + b_ref[...]
        if relu:
            o = jnp.maximum(o, 0.0)
        o_ref[...] = o
        s0 = jnp.sum(o, axis=0, keepdims=True)
        s1 = jnp.sum(o * o, axis=0, keepdims=True)
        st = jnp.concatenate([s0, s1], axis=0)

        @pl.when(i == 0)
        def _():
            st_ref[...] = st

        @pl.when(i > 0)
        def _():
            st_ref[...] += st

    return pl.pallas_call(
        body,
        grid=(NRB,),
        in_specs=[
            pl.BlockSpec((RB, H), lambda i: (i, 0)),
            pl.BlockSpec((1, 1, RB), lambda i: (i, 0, 0)),
            pl.BlockSpec((1, H), lambda i: (0, 0)),
        ],
        out_specs=[
            pl.BlockSpec((RB, H), lambda i: (i, 0)),
            pl.BlockSpec((2, H), lambda i: (0, 0)),
        ],
        out_shape=[
            jax.ShapeDtypeStruct((N, H), _F32),
            jax.ShapeDtypeStruct((2, H), _F32),
        ],
    )


def _post2_body(o_ref, st_ref, g_ref, be_ref, out_ref):
    s = st_ref[...]
    mu = s[0:1] / N
    var = s[1:2] / N - mu * mu
    inv = g_ref[...] * lax.rsqrt(var + 1e-5)
    out_ref[...] = (o_ref[...] - mu) * inv + be_ref[...]


_post2_call = pl.pallas_call(
    _post2_body,
    grid=(NRB,),
    in_specs=[
        pl.BlockSpec((RB, H), lambda i: (i, 0)),
        pl.BlockSpec((2, H), lambda i: (0, 0)),
        pl.BlockSpec((1, H), lambda i: (0, 0)),
        pl.BlockSpec((1, H), lambda i: (0, 0)),
    ],
    out_specs=pl.BlockSpec((RB, H), lambda i: (i, 0)),
    out_shape=jax.ShapeDtypeStruct((N, H), _F32),
)


def _pool_body(x_ref, bt_ref, gf_ref, w0, b0, w1, b1, w2, b2, w3, b3,
               out_ref, pooled, cnt):
    i = pl.program_id(0)

    @pl.when(i == 0)
    def _():
        pooled[...] = jnp.zeros((NB, H), _F32)
        cnt[...] = jnp.zeros((NB, 128), _F32)

    bt = bt_ref[0, 0, :]
    onehot = (bt[:, None] == lax.iota(jnp.int32, NB)[None, :]).astype(_F32)
    pooled[...] += lax.dot_general(onehot, x_ref[...], (((0,), (0,)), ((), ())),
                                   precision=_HIGH, preferred_element_type=_F32)
    cnt[...] += jnp.sum(onehot, axis=0)[:, None]

    @pl.when(i == NRB - 1)
    def _():
        xg = pooled[...] / jnp.maximum(cnt[:, 0:1], 1.0)
        x = jnp.concatenate([xg, gf_ref[...]], axis=1)
        x = jnp.maximum(_dot(x, w0[...]) + b0[0], 0.0)
        x = jnp.maximum(_dot(x, w1[...]) + b1[0], 0.0)
        x = jnp.maximum(_dot(x, w2[...]) + b2[0], 0.0)
        out_ref[...] = _dot(x, w3[...]) + b3[0]


_pool_call = pl.pallas_call(
    _pool_body,
    grid=(NRB,),
    in_specs=[
        pl.BlockSpec((RB, H), lambda i: (i, 0)),
        pl.BlockSpec((1, 1, RB), lambda i: (i, 0, 0)),
        pl.BlockSpec((NB, GFC), lambda i: (0, 0)),
        pl.BlockSpec((H + GFC, 64), lambda i: (0, 0)),
        pl.BlockSpec((1, 64), lambda i: (0, 0)),
        pl.BlockSpec((64, 16), lambda i: (0, 0)),
        pl.BlockSpec((1, 16), lambda i: (0, 0)),
        pl.BlockSpec((16, 8), lambda i: (0, 0)),
        pl.BlockSpec((1, 8), lambda i: (0, 0)),
        pl.BlockSpec((8, OUT), lambda i: (0, 0)),
        pl.BlockSpec((1, OUT), lambda i: (0, 0)),
    ],
    out_specs=pl.BlockSpec((NB, OUT), lambda i: (0, 0)),
    out_shape=jax.ShapeDtypeStruct((NB, OUT), _F32),
    scratch_shapes=[
        pltpu.VMEM((NB, H), _F32),
        pltpu.VMEM((NB, 128), _F32),
    ],
)


_pre1 = _make_pre([H])
_pre3 = _make_pre([H, H, H])
_post1_relu = _make_post1(True)
_post1_norelu = _make_post1(False)


def _conv(x_parts, s2, d2, ea, W, asrc, adst, We, aedge, b, gamma2, beta2,
          relu, after=None):
    A = jnp.concatenate(
        [asrc[:, None], adst[:, None], jnp.zeros((H, 126), _F32)], axis=1)
    pre = _pre3 if len(x_parts) == 3 else _pre1
    h4a, h4b, hsd = pre(*x_parts, W, A)
    et = _et_call(ea, We, aedge[:, None]).reshape(NT, NCH, K)
    if after is not None:
        et, _ = lax.optimization_barrier((et, after))
    hs = hsd[:, 0]
    hd = hsd[:, 1]
    num, den = _sc_gat(h4a, h4b, s2, d2, et, hs, hd)
    post1 = _post1_relu if relu else _post1_norelu
    o, st = post1(num, den, b[None])
    return _post2_call(o, st, gamma2, beta2), den


def kernel(x_alpha, x_beta, x_theta, edge_index_alpha, edge_index_beta,
           edge_index_theta, edge_attr_alpha, edge_attr_beta, edge_attr_theta,
           batch, global_feature,
           W_alpha, asrc_alpha, adst_alpha, We_alpha, aedge_alpha, b_alpha,
           W_beta, asrc_beta, adst_beta, We_beta, aedge_beta, b_beta,
           W_theta, asrc_theta, adst_theta, We_theta, aedge_theta, b_theta,
           W_combined, asrc_combined, adst_combined, We_combined,
           aedge_combined, b_combined,
           W_final, asrc_final, adst_final, We_final, aedge_final, b_final,
           bn_gamma, bn_beta,
           mlp_W0, mlp_b0, mlp_W1, mlp_b1, mlp_W2, mlp_b2, mlp_W3, mlp_b3):
    gamma2 = bn_gamma[None]
    beta2 = bn_beta[None]

    def split_ei(ei):
        return (ei[0].reshape(NT, NCH, K), ei[1].reshape(NT, NCH, K))

    sa, da = split_ei(edge_index_alpha)
    sb, db = split_ei(edge_index_beta)
    st_, dt = split_ei(edge_index_theta)

    xa, tok = _conv((x_alpha,), sa, da, edge_attr_alpha, W_alpha, asrc_alpha,
                    adst_alpha, We_alpha, aedge_alpha, b_alpha, gamma2, beta2,
                    True)
    xb, tok = _conv((x_beta,), sb, db, edge_attr_beta, W_beta, asrc_beta,
                    adst_beta, We_beta, aedge_beta, b_beta, gamma2, beta2,
                    True, after=tok)
    xt, tok = _conv((x_theta,), st_, dt, edge_attr_theta, W_theta, asrc_theta,
                    adst_theta, We_theta, aedge_theta, b_theta, gamma2, beta2,
                    True, after=tok)
    xf, tok = _conv((xa, xb, xt), sa, da, edge_attr_alpha, W_combined,
                    asrc_combined, adst_combined, We_combined, aedge_combined,
                    b_combined, gamma2, beta2, True, after=tok)
    xo, _ = _conv((xf,), sa, da, edge_attr_alpha, W_final, asrc_final,
                  adst_final, We_final, aedge_final, b_final, gamma2, beta2,
                  False, after=tok)

    batch3 = batch.reshape(NRB, 1, RB)
    return _pool_call(xo, batch3, global_feature, mlp_W0, mlp_b0[None],
                      mlp_W1, mlp_b1[None], mlp_W2, mlp_b2[None],
                      mlp_W3, mlp_b3[None])
